# Initial kernel scaffold; baseline (speedup 1.0000x reference)
import jax
import jax.numpy as jnp
from jax.experimental import pallas as pl


def kernel(pred_logits, pred_boxes, pred_corners, tgt_labels, tgt_boxes, tgt_corners):
    raise NotImplementedError("write your pallas kernel here")


# speedup vs baseline: 1.7755x; 1.7755x over previous
import jax
import jax.numpy as jnp
from jax import lax
from jax.experimental import pallas as pl
from jax.experimental.pallas import tpu as pltpu

_B, _Q, _NT, _C = 8, 256, 32, 32
_WCE, _WBB, _WGI = 1.0, 5.0, 2.0


def _loss_body(xT_ref, pbT_ref, pcT_ref, lbl_ref, tb_ref, tc_ref, out_ref):
    q_iota = lax.broadcasted_iota(jnp.int32, (1, _Q), 1)
    j_iota = lax.broadcasted_iota(jnp.int32, (_NT, 1), 0)
    ce_pos = jnp.float32(0.0)
    xz = jnp.float32(0.0)
    bbox = jnp.float32(0.0)
    giou_s = jnp.float32(0.0)
    for b in range(_B):
        x = xT_ref[b]
        ce_pos += jnp.sum(jnp.maximum(x, 0.0) + jnp.log(1.0 + jnp.exp(-jnp.abs(x))))
        lbl = lbl_ref[b]
        yT = jnp.zeros((_NT, _Q), jnp.float32)
        for c in range(_C):
            yT = yT + jnp.where(lbl == c, x[c : c + 1, :], 0.0)
        costT = -(1.0 / (1.0 + jnp.exp(-yT)))
        pb = pbT_ref[b]
        tb = tb_ref[b]
        cbb = jnp.zeros((_NT, _Q), jnp.float32)
        for dd in range(6):
            cbb = cbb + jnp.abs(pb[dd : dd + 1, :] - tb[:, dd : dd + 1])
        cost = costT + cbb

        def step(j, carry):
            used, st = carry
            ohj = j_iota == j
            row = jnp.sum(jnp.where(ohj, cost, 0.0), axis=0, keepdims=True)
            cvals = jnp.where(used > 0.5, jnp.inf, row)
            m = jnp.min(cvals)
            idx = jnp.min(jnp.where(cvals == m, q_iota, _Q))
            ohq = q_iota == idx
            used = jnp.maximum(used, jnp.where(ohq, 1.0, 0.0))
            st = st + jnp.where(jnp.logical_and(ohj, ohq), 1.0, 0.0)
            return used, st

        _, st = lax.fori_loop(
            0,
            _NT,
            step,
            (jnp.zeros((1, _Q), jnp.float32), jnp.zeros((_NT, _Q), jnp.float32)),
        )

        xz += jnp.sum(st * yT)
        bbox += jnp.sum(st * cbb)

        smin = pcT_ref[b, 0]
        smax = pcT_ref[b, 0]
        for k in range(1, 8):
            ck = pcT_ref[b, k]
            smin = jnp.minimum(smin, ck)
            smax = jnp.maximum(smax, ck)
        smin_m = jnp.concatenate(
            [jnp.sum(st * smin[dd : dd + 1, :], axis=1, keepdims=True) for dd in range(3)],
            axis=1,
        )
        smax_m = jnp.concatenate(
            [jnp.sum(st * smax[dd : dd + 1, :], axis=1, keepdims=True) for dd in range(3)],
            axis=1,
        )
        tmin = tc_ref[b, 0]
        tmax = tc_ref[b, 0]
        for k in range(1, 8):
            ck = tc_ref[b, k]
            tmin = jnp.minimum(tmin, ck)
            tmax = jnp.maximum(tmax, ck)
        e = jnp.maximum(jnp.minimum(smax_m, tmax) - jnp.maximum(smin_m, tmin), 0.0)
        inter = e[:, 0:1] * e[:, 1:2] * e[:, 2:3]
        ds_ = smax_m - smin_m
        vol_s = ds_[:, 0:1] * ds_[:, 1:2] * ds_[:, 2:3]
        dt_ = tmax - tmin
        vol_t = dt_[:, 0:1] * dt_[:, 1:2] * dt_[:, 2:3]
        union = vol_s + vol_t - inter
        ee = jnp.maximum(smax_m, tmax) - jnp.minimum(smin_m, tmin)
        enc = ee[:, 0:1] * ee[:, 1:2] * ee[:, 2:3]
        g = inter / (union + 1e-7) - (enc - union) / (enc + 1e-7)
        giou_s += jnp.sum(g)

    ce = (ce_pos - xz) / (_B * _Q * _C)
    bb = bbox / (_B * _NT * 6)
    gi = 1.0 - giou_s / (_B * _NT)
    out_ref[0] = ce * _WCE + bb * _WBB + gi * _WGI
    out_ref[1] = ce
    out_ref[2] = bb
    out_ref[3] = gi


def kernel(pred_logits, pred_boxes, pred_corners, tgt_labels, tgt_boxes, tgt_corners):
    xT = jnp.transpose(pred_logits, (0, 2, 1))
    pbT = jnp.transpose(pred_boxes, (0, 2, 1))
    pcT = jnp.transpose(pred_corners, (0, 2, 3, 1))
    lbl = tgt_labels.astype(jnp.int32).reshape(_B, _NT, 1)
    tc2 = jnp.transpose(tgt_corners, (0, 2, 1, 3))
    out = pl.pallas_call(
        _loss_body,
        out_shape=jax.ShapeDtypeStruct((4,), jnp.float32),
        out_specs=pl.BlockSpec(memory_space=pltpu.SMEM),
    )(xT, pbT, pcT, lbl, tgt_boxes, tc2)
    return (out[0], out[1], out[2], out[3])


# speedup vs baseline: 10.1697x; 5.7277x over previous
import jax
import jax.numpy as jnp
from jax import lax
from jax.experimental import pallas as pl
from jax.experimental.pallas import tpu as pltpu

_B, _Q, _NT, _C = 8, 256, 32, 32
_WCE, _WBB, _WGI = 1.0, 5.0, 2.0


def _loss_body(xT_ref, pbT_ref, pcT_ref, lbl_ref, tb_ref, tcT_ref, out_ref):
    x3 = xT_ref[...]
    ce_pos = jnp.sum(jnp.maximum(x3, 0.0) + jnp.log(1.0 + jnp.exp(-jnp.abs(x3))))

    lbl3 = lbl_ref[...]
    y3 = jnp.zeros((_B, _NT, _Q), jnp.float32)
    for c in range(_C):
        y3 = y3 + jnp.where(lbl3 == c, x3[:, c : c + 1, :], 0.0)

    pb3 = pbT_ref[...]
    tb3 = tb_ref[...]
    cbb3 = jnp.zeros((_B, _NT, _Q), jnp.float32)
    for dd in range(6):
        cbb3 = cbb3 + jnp.abs(pb3[:, dd : dd + 1, :] - tb3[:, :, dd : dd + 1])
    cost3 = -(1.0 / (1.0 + jnp.exp(-y3))) + cbb3

    smin = pcT_ref[:, 0]
    smax = pcT_ref[:, 0]
    for k in range(1, 8):
        ck = pcT_ref[:, k]
        smin = jnp.minimum(smin, ck)
        smax = jnp.maximum(smax, ck)

    q_iota = lax.broadcasted_iota(jnp.int32, (1, _Q), 1)
    used = jnp.zeros((_B, _Q), jnp.float32)
    xz_acc = jnp.zeros((_B, _Q), jnp.float32)
    bb_acc = jnp.zeros((_B, _Q), jnp.float32)
    sm_cols = [[] for _ in range(6)]
    for j in range(_NT):
        cvals = jnp.where(used > 0.5, jnp.inf, cost3[:, j, :])
        m = jnp.min(cvals, axis=1, keepdims=True)
        idx = jnp.min(jnp.where(cvals == m, q_iota, _Q), axis=1, keepdims=True)
        ohq = jnp.where(q_iota == idx, 1.0, 0.0)
        used = jnp.maximum(used, ohq)
        xz_acc = xz_acc + ohq * y3[:, j, :]
        bb_acc = bb_acc + ohq * cbb3[:, j, :]
        for dd in range(3):
            sm_cols[dd].append(jnp.sum(ohq * smin[:, dd, :], axis=1, keepdims=True))
            sm_cols[3 + dd].append(jnp.sum(ohq * smax[:, dd, :], axis=1, keepdims=True))

    xz = jnp.sum(xz_acc)
    bbox = jnp.sum(bb_acc)

    sminm = [jnp.concatenate(sm_cols[dd], axis=1) for dd in range(3)]
    smaxm = [jnp.concatenate(sm_cols[3 + dd], axis=1) for dd in range(3)]
    tmin, tmax = [], []
    for dd in range(3):
        lo = tcT_ref[:, dd, 0]
        hi = tcT_ref[:, dd, 0]
        for k in range(1, 8):
            ck = tcT_ref[:, dd, k]
            lo = jnp.minimum(lo, ck)
            hi = jnp.maximum(hi, ck)
        tmin.append(lo)
        tmax.append(hi)
    inter = jnp.float32(1.0)
    vol_s = jnp.float32(1.0)
    vol_t = jnp.float32(1.0)
    enc = jnp.float32(1.0)
    for dd in range(3):
        e = jnp.maximum(jnp.minimum(smaxm[dd], tmax[dd]) - jnp.maximum(sminm[dd], tmin[dd]), 0.0)
        inter = inter * e
        vol_s = vol_s * (smaxm[dd] - sminm[dd])
        vol_t = vol_t * (tmax[dd] - tmin[dd])
        enc = enc * (jnp.maximum(smaxm[dd], tmax[dd]) - jnp.minimum(sminm[dd], tmin[dd]))
    union = vol_s + vol_t - inter
    g = inter / (union + 1e-7) - (enc - union) / (enc + 1e-7)
    giou_s = jnp.sum(g)

    ce = (ce_pos - xz) / (_B * _Q * _C)
    bb = bbox / (_B * _NT * 6)
    gi = 1.0 - giou_s / (_B * _NT)
    out_ref[0] = ce * _WCE + bb * _WBB + gi * _WGI
    out_ref[1] = ce
    out_ref[2] = bb
    out_ref[3] = gi


def kernel(pred_logits, pred_boxes, pred_corners, tgt_labels, tgt_boxes, tgt_corners):
    xT = jnp.transpose(pred_logits, (0, 2, 1))
    pbT = jnp.transpose(pred_boxes, (0, 2, 1))
    pcT = jnp.transpose(pred_corners, (0, 2, 3, 1))
    lbl = tgt_labels.astype(jnp.int32).reshape(_B, _NT, 1)
    tcT = jnp.transpose(tgt_corners, (0, 3, 2, 1))
    out = pl.pallas_call(
        _loss_body,
        out_shape=jax.ShapeDtypeStruct((4,), jnp.float32),
        out_specs=pl.BlockSpec(memory_space=pltpu.SMEM),
    )(xT, pbT, pcT, lbl, tgt_boxes, tcT)
    return (out[0], out[1], out[2], out[3])


# speedup vs baseline: 14.2187x; 1.3982x over previous
import jax
import jax.numpy as jnp
from jax import lax
from jax.experimental import pallas as pl
from jax.experimental.pallas import tpu as pltpu

_B, _Q, _NT, _C = 8, 256, 32, 32
_WCE, _WBB, _WGI = 1.0, 5.0, 2.0


def _loss_body(xT_ref, pbT_ref, pcT_ref, lbl_ref, tb_ref, tcT_ref, out_ref):
    x3 = xT_ref[...]
    ce_pos = jnp.sum(jnp.maximum(x3, 0.0) + jnp.log(1.0 + jnp.exp(-jnp.abs(x3))))

    lbl3 = lbl_ref[...]
    y3 = jnp.zeros((_B, _NT, _Q), jnp.float32)
    for g in range(4):
        sub = jnp.clip(lbl3 - 8 * g, 0, 7)
        subB = jnp.broadcast_to(sub, (_B, _NT, _Q))
        part = jnp.take_along_axis(x3[:, 8 * g : 8 * g + 8, :], subB, axis=1)
        y3 = y3 + jnp.where((lbl3 >= 8 * g) & (lbl3 < 8 * g + 8), part, 0.0)

    pb3 = pbT_ref[...]
    tb3 = tb_ref[...]
    cbb3 = jnp.zeros((_B, _NT, _Q), jnp.float32)
    for dd in range(6):
        cbb3 = cbb3 + jnp.abs(pb3[:, dd : dd + 1, :] - tb3[:, :, dd : dd + 1])
    cost3 = -(1.0 / (1.0 + jnp.exp(-y3))) + cbb3

    costT = [jnp.transpose(cost3[:, j, :]) for j in range(_NT)]
    q_iota_s = lax.broadcasted_iota(jnp.int32, (_Q, 1), 0)
    usedT = jnp.zeros((_Q, _B), jnp.float32)
    rows = []
    for j in range(_NT):
        cv = jnp.where(usedT > 0.5, jnp.inf, costT[j])
        m = jnp.min(cv, axis=0, keepdims=True)
        idx = jnp.min(jnp.where(cv == m, q_iota_s, _Q), axis=0, keepdims=True)
        ohqT = jnp.where(q_iota_s == idx, 1.0, 0.0)
        usedT = jnp.maximum(usedT, ohqT)
        rows.append(jnp.transpose(ohqT).reshape(_B, 1, _Q))

    st3 = jnp.concatenate(rows, axis=1)
    xz = jnp.sum(st3 * y3)
    bbox = jnp.sum(st3 * cbb3)

    smin = pcT_ref[:, 0]
    smax = pcT_ref[:, 0]
    for k in range(1, 8):
        ck = pcT_ref[:, k]
        smin = jnp.minimum(smin, ck)
        smax = jnp.maximum(smax, ck)

    giou_s = jnp.float32(0.0)
    for b in range(_B):
        sm6 = jnp.concatenate([smin[b], smax[b]], axis=0)
        mm = lax.dot_general(
            sm6,
            st3[b],
            (((1,), (1,)), ((), ())),
            preferred_element_type=jnp.float32,
        )
        inter = jnp.float32(1.0)
        vol_s = jnp.float32(1.0)
        vol_t = jnp.float32(1.0)
        enc = jnp.float32(1.0)
        for dd in range(3):
            smn = mm[dd : dd + 1, :]
            smx = mm[3 + dd : 4 + dd, :]
            tmn = tcT_ref[b, dd, 0:1]
            tmx = tcT_ref[b, dd, 0:1]
            for k in range(1, 8):
                ck = tcT_ref[b, dd, k : k + 1]
                tmn = jnp.minimum(tmn, ck)
                tmx = jnp.maximum(tmx, ck)
            inter = inter * jnp.maximum(jnp.minimum(smx, tmx) - jnp.maximum(smn, tmn), 0.0)
            vol_s = vol_s * (smx - smn)
            vol_t = vol_t * (tmx - tmn)
            enc = enc * (jnp.maximum(smx, tmx) - jnp.minimum(smn, tmn))
        union = vol_s + vol_t - inter
        g = inter / (union + 1e-7) - (enc - union) / (enc + 1e-7)
        giou_s = giou_s + jnp.sum(g)

    ce = (ce_pos - xz) / (_B * _Q * _C)
    bb = bbox / (_B * _NT * 6)
    gi = 1.0 - giou_s / (_B * _NT)
    out_ref[0] = ce * _WCE + bb * _WBB + gi * _WGI
    out_ref[1] = ce
    out_ref[2] = bb
    out_ref[3] = gi


def kernel(pred_logits, pred_boxes, pred_corners, tgt_labels, tgt_boxes, tgt_corners):
    xT = jnp.transpose(pred_logits, (0, 2, 1))
    pbT = jnp.transpose(pred_boxes, (0, 2, 1))
    pcT = jnp.transpose(pred_corners, (0, 2, 3, 1))
    lbl = tgt_labels.astype(jnp.int32).reshape(_B, _NT, 1)
    tcT = jnp.transpose(tgt_corners, (0, 3, 2, 1))
    out = pl.pallas_call(
        _loss_body,
        out_shape=jax.ShapeDtypeStruct((4,), jnp.float32),
        out_specs=pl.BlockSpec(memory_space=pltpu.SMEM),
    )(xT, pbT, pcT, lbl, tgt_boxes, tcT)
    return (out[0], out[1], out[2], out[3])
